# baseline (speedup 1.0000x reference)
import jax
import jax.numpy as jnp
from jax import lax
from jax.experimental import pallas as pl
from jax.experimental.pallas import tpu as pltpu
from jax.experimental.pallas import tpu_sc as plsc

D_MODEL = 64
N_TOK = 4096 * 200
_INFO = plsc.get_sparse_core_info()
NC = _INFO.num_cores
NS = _INFO.num_subcores
NW = NC * NS
CH = 128
N_PER_W = N_TOK // NW
STEPS = N_PER_W // CH


def _embed_body(tok_hbm, tab_hbm, out_hbm, idx_v, rows_v, isem, gsem, ssem):
    wid = lax.axis_index("s") * NC + lax.axis_index("c")
    base = wid * N_PER_W

    def step(s, _):
        off = base + s * CH
        pltpu.async_copy(tok_hbm.at[pl.ds(off, CH)], idx_v, isem).wait()
        pltpu.async_copy(tab_hbm.at[idx_v], rows_v, gsem).wait()
        pltpu.async_copy(rows_v, out_hbm.at[pl.ds(off, CH)], ssem).wait()
        return 0

    lax.fori_loop(0, STEPS, step, 0)


@jax.jit
def kernel(tokens, W_E):
    B, S = tokens.shape
    flat = tokens.reshape(B * S)
    mesh = plsc.VectorSubcoreMesh(core_axis_name="c", subcore_axis_name="s")
    out = pl.kernel(
        _embed_body,
        mesh=mesh,
        out_type=jax.ShapeDtypeStruct((B * S, D_MODEL), jnp.float32),
        scratch_types=[
            pltpu.VMEM((CH,), jnp.int32),
            pltpu.VMEM((CH, D_MODEL), jnp.float32),
            pltpu.SemaphoreType.DMA,
            pltpu.SemaphoreType.DMA,
            pltpu.SemaphoreType.DMA,
        ],
        compiler_params=pltpu.CompilerParams(use_tc_tiling_on_sc=False),
    )(flat, W_E)
    return out.reshape(B, S, D_MODEL)


# speedup vs baseline: 1.1958x; 1.1958x over previous
import jax
import jax.numpy as jnp
from jax import lax
from jax.experimental import pallas as pl
from jax.experimental.pallas import tpu as pltpu
from jax.experimental.pallas import tpu_sc as plsc

D_MODEL = 64
N_TOK = 4096 * 200
_INFO = plsc.get_sparse_core_info()
NC = _INFO.num_cores
NS = _INFO.num_subcores
NW = NC * NS
GCH = 128
K = 4
CHUNK = K * GCH
N_PER_W = N_TOK // NW
STEPS = N_PER_W // CHUNK


def _embed_body(tok_hbm, tab_hbm, out_hbm, idx0, idx1, rows0, rows1,
                isem0, isem1, gsem0, gsem1, ssem0, ssem1):
    wid = lax.axis_index("s") * NC + lax.axis_index("c")
    base = wid * N_PER_W
    brow = wid * (N_PER_W // GCH)

    bufs = ((idx0, rows0, isem0, gsem0, ssem0),
            (idx1, rows1, isem1, gsem1, ssem1))

    def do_step(s, buf, wait_store, prefetch_s):
        idx_b, rows_b, isem_b, gsem_b, ssem_b = bufs[buf]
        if wait_store:
            pltpu.make_async_copy(rows_b, out_hbm.at[pl.ds(0, CHUNK)],
                                  ssem_b).wait()
        pltpu.make_async_copy(tok_hbm.at[pl.ds(0, K)], idx_b, isem_b).wait()
        for j in range(K):
            pltpu.async_copy(tab_hbm.at[idx_b.at[j]],
                             rows_b.at[pl.ds(j * GCH, GCH)], gsem_b)
        pltpu.make_async_copy(tab_hbm.at[pl.ds(0, CHUNK)], rows_b,
                              gsem_b).wait()
        if prefetch_s is not None:
            pltpu.async_copy(tok_hbm.at[pl.ds(brow + prefetch_s * K, K)],
                             idx_b, isem_b)
        pltpu.async_copy(rows_b, out_hbm.at[pl.ds(base + s * CHUNK, CHUNK)],
                         ssem_b)

    pltpu.async_copy(tok_hbm.at[pl.ds(brow, K)], idx0, isem0)
    pltpu.async_copy(tok_hbm.at[pl.ds(brow + K, K)], idx1, isem1)
    do_step(0, 0, False, 2)
    do_step(1, 1, False, 3)

    def pair(o, _):
        s = 2 * o
        do_step(s, 0, True, s + 2)
        do_step(s + 1, 1, True, s + 3)
        return 0

    lax.fori_loop(1, STEPS // 2 - 1, pair, 0)

    do_step(STEPS - 2, 0, True, None)
    do_step(STEPS - 1, 1, True, None)
    pltpu.make_async_copy(rows0, out_hbm.at[pl.ds(0, CHUNK)], ssem0).wait()
    pltpu.make_async_copy(rows1, out_hbm.at[pl.ds(0, CHUNK)], ssem1).wait()


@jax.jit
def kernel(tokens, W_E):
    B, S = tokens.shape
    tok2d = tokens.reshape(B * S // GCH, GCH)
    mesh = plsc.VectorSubcoreMesh(core_axis_name="c", subcore_axis_name="s")
    out = pl.kernel(
        _embed_body,
        mesh=mesh,
        out_type=jax.ShapeDtypeStruct((B * S, D_MODEL), jnp.float32),
        scratch_types=[
            pltpu.VMEM((K, GCH), jnp.int32),
            pltpu.VMEM((K, GCH), jnp.int32),
            pltpu.VMEM((CHUNK, D_MODEL), jnp.float32),
            pltpu.VMEM((CHUNK, D_MODEL), jnp.float32),
            pltpu.SemaphoreType.DMA,
            pltpu.SemaphoreType.DMA,
            pltpu.SemaphoreType.DMA,
            pltpu.SemaphoreType.DMA,
            pltpu.SemaphoreType.DMA,
            pltpu.SemaphoreType.DMA,
        ],
        compiler_params=pltpu.CompilerParams(use_tc_tiling_on_sc=False),
    )(tok2d, W_E)
    return out.reshape(B, S, D_MODEL)


# speedup vs baseline: 1.1959x; 1.0000x over previous
import jax
import jax.numpy as jnp
from jax import lax
from jax.experimental import pallas as pl
from jax.experimental.pallas import tpu as pltpu
from jax.experimental.pallas import tpu_sc as plsc

D_MODEL = 64
N_TOK = 4096 * 200
_INFO = plsc.get_sparse_core_info()
NC = _INFO.num_cores
NS = _INFO.num_subcores
NW = NC * NS
GCH = 128
K = 4
CHUNK = K * GCH
N_PER_W = N_TOK // NW
STEPS = N_PER_W // CHUNK


def _embed_body(tok_hbm, tab_hbm, out_hbm, idx0, idx1, rows0, rows1,
                isem0, isem1, gsem0, gsem1, ssem0, ssem1):
    wid = lax.axis_index("s") * NC + lax.axis_index("c")
    base = wid * N_PER_W
    brow = wid * (N_PER_W // GCH)

    bufs = ((idx0, rows0, isem0, gsem0, ssem0),
            (idx1, rows1, isem1, gsem1, ssem1))

    def fire(buf):
        idx_b, rows_b, _, gsem_b, _ = bufs[buf]
        for j in range(K):
            pltpu.async_copy(tab_hbm.at[idx_b.at[j]],
                             rows_b.at[pl.ds(j * GCH, GCH)], gsem_b)

    def do_step(s, buf, wait_store_prev, fire_next, has_prefetch):
        idx_b, rows_b, isem_b, gsem_b, ssem_b = bufs[buf]
        idx_o, rows_o, isem_o, gsem_o, ssem_o = bufs[1 - buf]
        if fire_next:
            if wait_store_prev:
                pltpu.make_async_copy(rows_o, out_hbm.at[pl.ds(0, CHUNK)],
                                      ssem_o).wait()
            pltpu.make_async_copy(tok_hbm.at[pl.ds(0, K)], idx_o,
                                  isem_o).wait()
            fire(1 - buf)
        pltpu.make_async_copy(tab_hbm.at[pl.ds(0, CHUNK)], rows_b,
                              gsem_b).wait()
        if has_prefetch:
            pltpu.async_copy(tok_hbm.at[pl.ds(brow + (s + 2) * K, K)],
                             idx_b, isem_b)
        pltpu.async_copy(rows_b, out_hbm.at[pl.ds(base + s * CHUNK, CHUNK)],
                         ssem_b)

    pltpu.async_copy(tok_hbm.at[pl.ds(brow, K)], idx0, isem0)
    pltpu.async_copy(tok_hbm.at[pl.ds(brow + K, K)], idx1, isem1)
    pltpu.make_async_copy(tok_hbm.at[pl.ds(0, K)], idx0, isem0).wait()
    fire(0)
    do_step(0, 0, False, True, True)
    do_step(1, 1, True, True, True)

    def pair(o, _):
        s = 2 * o
        do_step(s, 0, True, True, True)
        do_step(s + 1, 1, True, True, True)
        return 0

    lax.fori_loop(1, STEPS // 2 - 1, pair, 0)

    do_step(STEPS - 2, 0, True, True, False)
    do_step(STEPS - 1, 1, False, False, False)
    pltpu.make_async_copy(rows0, out_hbm.at[pl.ds(0, CHUNK)], ssem0).wait()
    pltpu.make_async_copy(rows1, out_hbm.at[pl.ds(0, CHUNK)], ssem1).wait()


@jax.jit
def kernel(tokens, W_E):
    B, S = tokens.shape
    tok2d = tokens.reshape(B * S // GCH, GCH)
    mesh = plsc.VectorSubcoreMesh(core_axis_name="c", subcore_axis_name="s")
    out = pl.kernel(
        _embed_body,
        mesh=mesh,
        out_type=jax.ShapeDtypeStruct((B * S, D_MODEL), jnp.float32),
        scratch_types=[
            pltpu.VMEM((K, GCH), jnp.int32),
            pltpu.VMEM((K, GCH), jnp.int32),
            pltpu.VMEM((CHUNK, D_MODEL), jnp.float32),
            pltpu.VMEM((CHUNK, D_MODEL), jnp.float32),
            pltpu.SemaphoreType.DMA,
            pltpu.SemaphoreType.DMA,
            pltpu.SemaphoreType.DMA,
            pltpu.SemaphoreType.DMA,
            pltpu.SemaphoreType.DMA,
            pltpu.SemaphoreType.DMA,
        ],
        compiler_params=pltpu.CompilerParams(use_tc_tiling_on_sc=False),
    )(tok2d, W_E)
    return out.reshape(B, S, D_MODEL)


# speedup vs baseline: 1.4582x; 1.2193x over previous
import jax
import jax.numpy as jnp
from jax import lax
from jax.experimental import pallas as pl
from jax.experimental.pallas import tpu as pltpu
from jax.experimental.pallas import tpu_sc as plsc

D_PAD = 128
N_TOK = 4096 * 200
_INFO = plsc.get_sparse_core_info()
NC = _INFO.num_cores
NS = _INFO.num_subcores
NW = NC * NS
GCH = 128
K = 2
CHUNK = K * GCH
N_PER_W = N_TOK // NW
STEPS = N_PER_W // CHUNK


def _embed_body(tok_hbm, tab_hbm, out_hbm, idx0, idx1, rows0, rows1,
                isem0, isem1, gsem0, gsem1, ssem0, ssem1):
    wid = lax.axis_index("s") * NC + lax.axis_index("c")
    base = wid * N_PER_W

    bufs = ((idx0, rows0, isem0, gsem0, ssem0),
            (idx1, rows1, isem1, gsem1, ssem1))

    def fire(buf):
        idx_b, rows_b, _, gsem_b, _ = bufs[buf]
        for j in range(K):
            pltpu.async_copy(tab_hbm.at[idx_b.at[pl.ds(j * GCH, GCH)]],
                             rows_b.at[pl.ds(j * GCH, GCH)], gsem_b)

    def do_step(s, buf, wait_store_prev, fire_next, has_prefetch):
        idx_b, rows_b, isem_b, gsem_b, ssem_b = bufs[buf]
        idx_o, rows_o, isem_o, gsem_o, ssem_o = bufs[1 - buf]
        if fire_next:
            if wait_store_prev:
                pltpu.make_async_copy(rows_o, out_hbm.at[pl.ds(0, CHUNK)],
                                      ssem_o).wait()
            pltpu.make_async_copy(tok_hbm.at[pl.ds(0, CHUNK)], idx_o,
                                  isem_o).wait()
            fire(1 - buf)
        pltpu.make_async_copy(tab_hbm.at[pl.ds(0, CHUNK)], rows_b,
                              gsem_b).wait()
        if has_prefetch:
            pltpu.async_copy(tok_hbm.at[pl.ds(base + (s + 2) * CHUNK, CHUNK)],
                             idx_b, isem_b)
        pltpu.async_copy(rows_b, out_hbm.at[pl.ds(base + s * CHUNK, CHUNK)],
                         ssem_b)

    pltpu.async_copy(tok_hbm.at[pl.ds(base, CHUNK)], idx0, isem0)
    pltpu.async_copy(tok_hbm.at[pl.ds(base + CHUNK, CHUNK)], idx1, isem1)
    pltpu.make_async_copy(tok_hbm.at[pl.ds(0, CHUNK)], idx0, isem0).wait()
    fire(0)
    do_step(0, 0, False, True, True)
    do_step(1, 1, True, True, True)

    def pair(o, _):
        s = 2 * o
        do_step(s, 0, True, True, True)
        do_step(s + 1, 1, True, True, True)
        return 0

    lax.fori_loop(1, STEPS // 2 - 1, pair, 0)

    do_step(STEPS - 2, 0, True, True, False)
    do_step(STEPS - 1, 1, False, False, False)
    pltpu.make_async_copy(rows0, out_hbm.at[pl.ds(0, CHUNK)], ssem0).wait()
    pltpu.make_async_copy(rows1, out_hbm.at[pl.ds(0, CHUNK)], ssem1).wait()


@jax.jit
def kernel(tokens, W_E):
    B, S = tokens.shape
    V, D = W_E.shape
    tok1d = tokens.reshape(B * S)
    W2 = jnp.pad(W_E, ((0, 0), (0, D_PAD - D)))
    mesh = plsc.VectorSubcoreMesh(core_axis_name="c", subcore_axis_name="s")
    outp = pl.kernel(
        _embed_body,
        mesh=mesh,
        out_type=jax.ShapeDtypeStruct((B * S, D_PAD), jnp.float32),
        scratch_types=[
            pltpu.VMEM((CHUNK,), jnp.int32),
            pltpu.VMEM((CHUNK,), jnp.int32),
            pltpu.VMEM((CHUNK, D_PAD), jnp.float32),
            pltpu.VMEM((CHUNK, D_PAD), jnp.float32),
            pltpu.SemaphoreType.DMA,
            pltpu.SemaphoreType.DMA,
            pltpu.SemaphoreType.DMA,
            pltpu.SemaphoreType.DMA,
            pltpu.SemaphoreType.DMA,
            pltpu.SemaphoreType.DMA,
        ],
        compiler_params=pltpu.CompilerParams(use_tc_tiling_on_sc=True),
    )(tok1d, W2)
    return outp[:, :D].reshape(B, S, D)
